# baseline (speedup 1.0000x reference)
import functools

import jax
import jax.numpy as jnp
from jax import lax
from jax.experimental import pallas as pl
from jax.experimental.pallas import tpu as pltpu
from jax.experimental.pallas import tpu_sc as plsc

N_NODES = 10000
H = 100
E = 320000
NB = 4
NREL = 474

YB = 128
YW = NB * YB
AW = 112
CHUNK = 16
TILES = 16
EDGES_PER_TILE = E // 2 // TILES
NCHUNK = EDGES_PER_TILE // CHUNK

ZR_A = 632
ZR_LAST = N_NODES - (TILES - 1) * ZR_A
ZBUF = 104

_AWIN = tuple(range(0, AW, 16))


def _tc_basis_transform(x, basis_pad):
    BLK = 2000

    def body(x_ref, b_ref, o_ref):
        for b in range(NB):
            o_ref[:, pl.ds(b * YB, YB)] = jnp.dot(
                x_ref[...], b_ref[b], preferred_element_type=jnp.float32)

    return pl.pallas_call(
        body,
        grid=(N_NODES // BLK,),
        in_specs=[
            pl.BlockSpec((BLK, H), lambda i: (i, 0)),
            pl.BlockSpec((NB, H, YB), lambda i: (0, 0, 0)),
        ],
        out_specs=pl.BlockSpec((BLK, YW), lambda i: (i, 0)),
        out_shape=jax.ShapeDtypeStruct((N_NODES, YW), jnp.float32),
    )(x, basis_pad)


def _sc_accumulate(y, src, dst, rel, norm_flat, w_flat):
    mesh = plsc.VectorSubcoreMesh(core_axis_name="c", subcore_axis_name="s")

    @functools.partial(
        pl.kernel,
        mesh=mesh,
        out_type=jax.ShapeDtypeStruct((2, N_NODES, AW), jnp.float32),
        compiler_params=pltpu.CompilerParams(
            needs_layout_passes=False, use_tc_tiling_on_sc=False),
        scratch_types=[
            pltpu.VMEM_SHARED((N_NODES, AW), jnp.float32),
            pltpu.VMEM((NREL * NB,), jnp.float32),
            pltpu.VMEM((2, CHUNK), jnp.int32),
            pltpu.VMEM((2, CHUNK), jnp.int32),
            pltpu.VMEM((2, CHUNK), jnp.int32),
            pltpu.VMEM((2, CHUNK), jnp.float32),
            pltpu.VMEM((2, CHUNK), jnp.int32),
            pltpu.VMEM((2, CHUNK), jnp.float32),
            pltpu.VMEM((2, CHUNK), jnp.float32),
            pltpu.VMEM((2, CHUNK), jnp.float32),
            pltpu.VMEM((2, CHUNK), jnp.float32),
            pltpu.VMEM((2, CHUNK, YW), jnp.float32),
            pltpu.VMEM((2, CHUNK, AW), jnp.float32),
            pltpu.VMEM((ZBUF, AW), jnp.float32),
            (pltpu.SemaphoreType.DMA, pltpu.SemaphoreType.DMA),
            (pltpu.SemaphoreType.DMA, pltpu.SemaphoreType.DMA),
            (pltpu.SemaphoreType.DMA, pltpu.SemaphoreType.DMA),
        ],
    )
    def k(y_hbm, src_hbm, dst_hbm, r_hbm, norm_hbm, w_hbm, out_hbm,
          acc, w_v, srcb, dstb, relb, normb, sdst, c0b, c1b, c2b, c3b,
          rowsb, zb, z_v, msem, gsem, ssem):
        c = lax.axis_index("c")
        s = lax.axis_index("s")

        pltpu.sync_copy(w_hbm, w_v)

        zv = jnp.zeros((16,), jnp.float32)

        def zrow(i, carry):
            for off in _AWIN:
                z_v[i, pl.ds(off, 16)] = zv
            return carry

        lax.fori_loop(0, ZBUF, zrow, 0)

        rr = s * ZR_A
        ebase = c * (E // 2) + s * EDGES_PER_TILE

        def issue_meta(i, par):
            base = ebase + i * CHUNK
            pltpu.async_copy(src_hbm.at[pl.ds(base, CHUNK)], srcb.at[par], msem[par])
            pltpu.async_copy(dst_hbm.at[pl.ds(base, CHUNK)], dstb.at[par], msem[par])
            pltpu.async_copy(r_hbm.at[pl.ds(base, CHUNK)], relb.at[par], msem[par])
            pltpu.async_copy(norm_hbm.at[pl.ds(base, CHUNK)], normb.at[par], msem[par])

        def wait_meta(par):
            pltpu.make_async_copy(src_hbm.at[pl.ds(0, CHUNK)], srcb.at[par], msem[par]).wait()
            pltpu.make_async_copy(dst_hbm.at[pl.ds(0, CHUNK)], dstb.at[par], msem[par]).wait()
            pltpu.make_async_copy(r_hbm.at[pl.ds(0, CHUNK)], relb.at[par], msem[par]).wait()
            pltpu.make_async_copy(norm_hbm.at[pl.ds(0, CHUNK)], normb.at[par],
                                  msem[par]).wait()

        for blk in range(5):
            pltpu.sync_copy(z_v, acc.at[pl.ds(rr + blk * ZBUF, ZBUF)])

        @pl.when(s < TILES - 1)
        def _():
            pltpu.sync_copy(z_v, acc.at[pl.ds(rr + 5 * ZBUF, ZBUF)])
            pltpu.sync_copy(z_v.at[pl.ds(0, 8)], acc.at[pl.ds(rr + 624, 8)])

        plsc.subcore_barrier()

        issue_meta(0, 0)
        issue_meta(1, 1)
        wait_meta(0)
        pltpu.async_copy(y_hbm.at[srcb.at[0]], rowsb.at[0], gsem[0])

        def scatter_wait(par):
            pltpu.make_async_copy(zb.at[par], acc.at[sdst.at[par]], ssem[par]).wait()

        def step(kk, i, par):
            @pl.when(i + 1 < NCHUNK)
            def _():
                wait_meta(1 - par)
                pltpu.async_copy(y_hbm.at[srcb.at[1 - par]], rowsb.at[1 - par],
                                 gsem[1 - par])

            @pl.when(kk >= 1)
            def _():
                scatter_wait(par)

            rv = relb[par, pl.ds(0, CHUNK)]
            nv = normb[par, pl.ds(0, CHUNK)]
            i0 = rv * NB
            c0b[par, pl.ds(0, CHUNK)] = plsc.load_gather(w_v, [i0]) * nv
            c1b[par, pl.ds(0, CHUNK)] = plsc.load_gather(w_v, [i0 + 1]) * nv
            c2b[par, pl.ds(0, CHUNK)] = plsc.load_gather(w_v, [i0 + 2]) * nv
            c3b[par, pl.ds(0, CHUNK)] = plsc.load_gather(w_v, [i0 + 3]) * nv
            sdst[par, pl.ds(0, CHUNK)] = dstb[par, pl.ds(0, CHUNK)]

            pltpu.make_async_copy(y_hbm.at[srcb.at[par]], rowsb.at[par],
                                  gsem[par]).wait()

            c0g = c0b[par, pl.ds(0, CHUNK)]
            c1g = c1b[par, pl.ds(0, CHUNK)]
            c2g = c2b[par, pl.ds(0, CHUNK)]
            c3g = c3b[par, pl.ds(0, CHUNK)]
            for e in range(CHUNK):
                f0 = c0g[e]
                f1 = c1g[e]
                f2 = c2g[e]
                f3 = c3g[e]
                for off in _AWIN:
                    v = (rowsb[par, e, pl.ds(off, 16)] * f0
                         + rowsb[par, e, pl.ds(YB + off, 16)] * f1
                         + rowsb[par, e, pl.ds(2 * YB + off, 16)] * f2
                         + rowsb[par, e, pl.ds(3 * YB + off, 16)] * f3)
                    zb[par, e, pl.ds(off, 16)] = v

            pltpu.async_copy(zb.at[par], acc.at[sdst.at[par]], ssem[par],
                             add=True)

            @pl.when(i + 2 < NCHUNK)
            def _():
                issue_meta(i + 2, par)

        def pipe(kk, carry):
            step(kk, 2 * kk, 0)
            step(kk, 2 * kk + 1, 1)
            return carry

        lax.fori_loop(0, NCHUNK // 2, pipe, 0)
        step(NCHUNK // 2, NCHUNK - 1, 0)
        scatter_wait(1)
        scatter_wait(0)
        plsc.subcore_barrier()

        @pl.when(s < TILES - 1)
        def _():
            pltpu.sync_copy(acc.at[pl.ds(rr, ZR_A)],
                            out_hbm.at[c, pl.ds(rr, ZR_A)])

        @pl.when(s == TILES - 1)
        def _():
            pltpu.sync_copy(acc.at[pl.ds(rr, ZR_LAST)],
                            out_hbm.at[c, pl.ds(rr, ZR_LAST)])

    return k(y, src, dst, rel, norm_flat, w_flat)


def _tc_combine(part, x, loop_weight, h_bias2d):
    BLK = 2000

    def body(p_ref, x_ref, lw_ref, bias_ref, o_ref):
        out = jnp.dot(x_ref[...], lw_ref[...], preferred_element_type=jnp.float32)
        o_ref[...] = out + p_ref[0, :, :H] + p_ref[1, :, :H] + bias_ref[...]

    return pl.pallas_call(
        body,
        grid=(N_NODES // BLK,),
        in_specs=[
            pl.BlockSpec((2, BLK, AW), lambda i: (0, i, 0)),
            pl.BlockSpec((BLK, H), lambda i: (i, 0)),
            pl.BlockSpec((H, H), lambda i: (0, 0)),
            pl.BlockSpec((1, H), lambda i: (0, 0)),
        ],
        out_specs=pl.BlockSpec((BLK, H), lambda i: (i, 0)),
        out_shape=jax.ShapeDtypeStruct((N_NODES, H), jnp.float32),
    )(part, x, loop_weight, h_bias2d)


def kernel(h, edge_index, r, norm, emb_table, basis, w_comp, loop_weight, h_bias):
    x = jnp.take(emb_table, h, axis=0)
    basis_pad = jnp.pad(basis, ((0, 0), (0, 0), (0, YB - H)))
    y = _tc_basis_transform(x, basis_pad)
    part = _sc_accumulate(y, edge_index[0], edge_index[1], r,
                          norm.reshape(-1), w_comp.reshape(-1))
    return _tc_combine(part, x, loop_weight, h_bias.reshape(1, H))
